# baseline (speedup 1.0000x reference)
import functools

import jax
import jax.numpy as jnp
from jax import lax
from jax.experimental import pallas as pl
from jax.experimental.pallas import tpu as pltpu
from jax.experimental.pallas import tpu_sc as plsc

VOCAB = 1000000
T = 200
D = 32
B = 1024
N = B * T

NC, NS, L = 2, 16, 16
NW = NC * NS
PER_W = N // NW
CHUNK = 1600
NCH = PER_W // CHUNK
REP = CHUNK // T
HALVES = D // L


def _body(tok_hbm, pos_hbm, x_hbm, out_hbm, idx_v, pos_v, rows_v, sem):
    wid = lax.axis_index("s") * NC + lax.axis_index("c")
    base = wid * PER_W

    pltpu.sync_copy(x_hbm.at[pl.ds(base, PER_W)], idx_v)
    pltpu.sync_copy(pos_hbm, pos_v)

    for c in range(NCH):
        row0 = c * CHUNK
        pltpu.async_copy(
            tok_hbm.at[idx_v.at[pl.ds(row0, CHUNK)]], rows_v, sem
        ).wait()

        def add_pos(t, _):
            for h in range(HALVES):
                pv = pos_v[t, pl.ds(h * L, L)]
                for r in range(REP):
                    j = r * T + t
                    rows_v[j, pl.ds(h * L, L)] = (
                        rows_v[j, pl.ds(h * L, L)] + pv
                    )
            return 0

        lax.fori_loop(0, T, add_pos, 0)

        pltpu.sync_copy(rows_v, out_hbm.at[pl.ds(base + row0, CHUNK)])


_mesh = plsc.VectorSubcoreMesh(core_axis_name="c", subcore_axis_name="s")

_embed = functools.partial(
    pl.kernel,
    out_type=jax.ShapeDtypeStruct((N, D), jnp.float32),
    mesh=_mesh,
    scratch_types=[
        pltpu.VMEM((PER_W,), jnp.int32),
        pltpu.VMEM((T, D), jnp.float32),
        pltpu.VMEM((CHUNK, D), jnp.float32),
        pltpu.SemaphoreType.DMA,
    ],
    compiler_params=pltpu.CompilerParams(use_tc_tiling_on_sc=False),
)(_body)


def kernel(token_table, pos_table, x):
    x_flat = x.reshape(-1).astype(jnp.int32)
    out = _embed(token_table, pos_table, x_flat)
    return out.reshape(B, T, D)


# speedup vs baseline: 1.0125x; 1.0125x over previous
import functools

import jax
import jax.numpy as jnp
from jax import lax
from jax.experimental import pallas as pl
from jax.experimental.pallas import tpu as pltpu
from jax.experimental.pallas import tpu_sc as plsc

VOCAB = 1000000
T = 200
D = 32
B = 1024
N = B * T

NC, NS, L = 2, 16, 16
NW = NC * NS
PER_W = N // NW
CHUNK = 800
NCH = PER_W // CHUNK
REP = CHUNK // T
HALVES = D // L
NB = 4


def _body(tok_hbm, pos_hbm, x_hbm, out_hbm, idx_v, pos_v,
          rows0, rows1, rows2, rows3,
          g0, g1, g2, g3, s0, s1, s2, s3):
    wid = lax.axis_index("s") * NC + lax.axis_index("c")
    base = wid * PER_W

    pltpu.sync_copy(x_hbm.at[pl.ds(base, PER_W)], idx_v)
    pltpu.sync_copy(pos_hbm, pos_v)

    bufs = (rows0, rows1, rows2, rows3)
    gsems = (g0, g1, g2, g3)
    ssems = (s0, s1, s2, s3)

    def start_gather(c):
        return pltpu.async_copy(
            tok_hbm.at[idx_v.at[pl.ds(c * CHUNK, CHUNK)]],
            bufs[c % NB], gsems[c % NB])

    def start_store(c):
        return pltpu.async_copy(
            bufs[c % NB], out_hbm.at[pl.ds(base + c * CHUNK, CHUNK)],
            ssems[c % NB])

    def add_pos(buf):
        def body_t(t, _):
            for h in range(HALVES):
                pv = pos_v[t, pl.ds(h * L, L)]
                for r in range(REP):
                    j = r * T + t
                    buf[j, pl.ds(h * L, L)] = buf[j, pl.ds(h * L, L)] + pv
            return 0
        lax.fori_loop(0, T, body_t, 0)

    gd = [None] * NCH
    sd = [None] * NCH
    gd[0] = start_gather(0)
    gd[1] = start_gather(1)
    for c in range(NCH):
        gd[c].wait()
        if c + 2 < NCH:
            if c - 2 >= 0:
                sd[c - 2].wait()
            gd[c + 2] = start_gather(c + 2)
        add_pos(bufs[c % NB])
        sd[c] = start_store(c)
    for c in range(max(0, NCH - NB), NCH):
        sd[c].wait()


_mesh = plsc.VectorSubcoreMesh(core_axis_name="c", subcore_axis_name="s")

_embed = functools.partial(
    pl.kernel,
    out_type=jax.ShapeDtypeStruct((N, D), jnp.float32),
    mesh=_mesh,
    scratch_types=(
        [pltpu.VMEM((PER_W,), jnp.int32),
         pltpu.VMEM((T, D), jnp.float32)]
        + [pltpu.VMEM((CHUNK, D), jnp.float32) for _ in range(NB)]
        + [pltpu.SemaphoreType.DMA for _ in range(2 * NB)]
    ),
    compiler_params=pltpu.CompilerParams(use_tc_tiling_on_sc=False),
)(_body)


def kernel(token_table, pos_table, x):
    x_flat = x.reshape(-1).astype(jnp.int32)
    out = _embed(token_table, pos_table, x_flat)
    return out.reshape(B, T, D)
